# baseline (speedup 1.0000x reference)
import functools

import jax
import jax.numpy as jnp
from jax import lax
from jax.experimental import pallas as pl
from jax.experimental.pallas import tpu as pltpu
from jax.experimental.pallas import tpu_sc as plsc

_N_TASKS = 1000000
_EMBED = 16
_HIST = 200
_SEQ = 190
_OUT_COLS = _EMBED + 1
_CH = 8
_N_CHUNKS = _HIST // _CH
_NW = 16


def _chunk_ids_marks(base, st0_v, st1_v, null_b, lane):
    in_hist = (base + lane) < _SEQ
    ids = jnp.where(in_hist, st0_v[pl.ds(base, 16)].astype(jnp.int32), _N_TASKS)
    marks = jnp.where(in_hist, st1_v[pl.ds(base, 16)], null_b)
    return ids, marks


def _fire_gathers(k, ids, col_v, tableT_hbm, sem, lane):
    copies = []
    for i in range(_CH):
        row_id = jnp.sum(jnp.where(lane == i, ids, 0))
        tile_base = pl.multiple_of(lax.shift_right_logical(row_id, 7) * 128, 128)
        copies.append(
            pltpu.make_async_copy(
                tableT_hbm.at[:, pl.ds(tile_base, 128)], col_v.at[k * _CH + i], sem
            )
        )
        copies[-1].start()
    return copies


def _assemble_chunk(k, ids, marks, col_v, out_v, lane):
    offs = jnp.bitwise_and(ids, 127)
    for i in range(_CH):
        off_b = jnp.sum(jnp.where(lane == i, offs, 0)) + jnp.zeros((16,), jnp.int32)
        out_v[pl.ds((k * _CH + i) * _OUT_COLS, _EMBED)] = plsc.load_gather(
            col_v, [jnp.full((16,), k * _CH + i, jnp.int32), lane, off_b]
        )
    mcol = k * _CH * _OUT_COLS + jnp.minimum(
        lane * _OUT_COLS + _EMBED, _CH * _OUT_COLS - 1
    )
    plsc.store_scatter(out_v, [mcol], marks, mask=lane < _CH)


def _run_chunks(bases, waits, st0_v, st1_v, null_v, col_v, out_v, tableT_hbm, out_hbm, sem):
    lane = lax.iota(jnp.int32, 16)
    ids_l, copies = [], []
    for k, b in enumerate(bases):
        in_hist = (b + lane) < _SEQ
        ids = jnp.where(in_hist, st0_v[pl.ds(b, 16)].astype(jnp.int32), _N_TASKS)
        ids_l.append(ids)
        copies.extend(_fire_gathers(k, ids, col_v, tableT_hbm, sem, lane))
    for w in waits:
        w.wait()
    null_b = jnp.sum(jnp.where(lane == 0, null_v[pl.ds(0, 16)], 0.0)) + jnp.zeros(
        (16,), jnp.float32
    )
    marks_l = [
        jnp.where((b + lane) < _SEQ, st1_v[pl.ds(b, 16)], null_b) for b in bases
    ]
    for c in copies:
        c.wait()
    outs = []
    for k, (ids, marks) in enumerate(zip(ids_l, marks_l)):
        _assemble_chunk(k, ids, marks, col_v, out_v, lane)
        w = pltpu.make_async_copy(
            out_v.at[pl.ds(k * _CH * _OUT_COLS, _CH * _OUT_COLS)],
            out_hbm.at[pl.ds(bases[k] * _OUT_COLS, _CH * _OUT_COLS)],
            sem,
        )
        w.start()
        outs.append(w)
    for w in outs:
        w.wait()


def _body(st0_hbm, st1_hbm, tableT_hbm, null_hbm, out_hbm, st0_v, st1_v, null_v, col_v, out_v, sem, sem_in):
    wid = lax.axis_index("s")

    c0 = pltpu.make_async_copy(st0_hbm, st0_v.at[pl.ds(0, _SEQ)], sem_in)
    c1 = pltpu.make_async_copy(st1_hbm, st1_v.at[pl.ds(0, _SEQ)], sem_in)
    c2 = pltpu.make_async_copy(null_hbm.at[0], null_v.at[pl.ds(0, 1)], sem_in)
    c0.start()
    c1.start()
    c2.start()
    c0.wait()

    args = (st0_v, st1_v, null_v, col_v, out_v, tableT_hbm, out_hbm, sem)

    @pl.when(wid < _N_CHUNKS - _NW)
    def _():
        _run_chunks([wid * _CH, (wid + _NW) * _CH], [c1, c2], *args)

    @pl.when(wid >= _N_CHUNKS - _NW)
    def _():
        _run_chunks([wid * _CH], [c1, c2], *args)


@functools.partial(
    pl.kernel,
    out_type=jax.ShapeDtypeStruct((_HIST * _OUT_COLS,), jnp.float32),
    mesh=plsc.VectorSubcoreMesh(
        core_axis_name="c", subcore_axis_name="s", num_cores=1
    ),
    scratch_types=[
        pltpu.VMEM((_HIST + 8,), jnp.float32),
        pltpu.VMEM((_HIST + 8,), jnp.float32),
        pltpu.VMEM((16,), jnp.float32),
        pltpu.VMEM((2 * _CH, _EMBED, 128), jnp.float32),
        pltpu.VMEM((2 * _CH * _OUT_COLS,), jnp.float32),
        pltpu.SemaphoreType.DMA,
        pltpu.SemaphoreType.DMA,
    ],
    compiler_params=pltpu.CompilerParams(
        needs_layout_passes=False, use_tc_tiling_on_sc=True
    ),
)
def _embed_gather(st0_hbm, st1_hbm, tableT_hbm, null_hbm, out_hbm, st0_v, st1_v, null_v, col_v, out_v, sem, sem_in):
    _body(st0_hbm, st1_hbm, tableT_hbm, null_hbm, out_hbm, st0_v, st1_v, null_v, col_v, out_v, sem, sem_in)


def kernel(st, task_table, null_mark_table):
    out = _embed_gather(st[0], st[1], task_table.T, null_mark_table)
    return out.reshape(1, _HIST, _OUT_COLS)


# speedup vs baseline: 1.0048x; 1.0048x over previous
import functools

import jax
import jax.numpy as jnp
from jax import lax
from jax.experimental import pallas as pl
from jax.experimental.pallas import tpu as pltpu
from jax.experimental.pallas import tpu_sc as plsc

_N_TASKS = 1000000
_EMBED = 16
_HIST = 200
_SEQ = 190
_OUT_COLS = _EMBED + 1
_CH = 8
_N_CHUNKS = _HIST // _CH
_NW = 16


def _chunk_ids_marks(base, st0_v, st1_v, null_b, lane):
    in_hist = (base + lane) < _SEQ
    ids = jnp.where(in_hist, st0_v[pl.ds(base, 16)].astype(jnp.int32), _N_TASKS)
    marks = jnp.where(in_hist, st1_v[pl.ds(base, 16)], null_b)
    return ids, marks


def _fire_gathers(k, ids, col_v, tableT_hbm, sem, lane):
    copies = []
    for i in range(_CH):
        row_id = jnp.sum(jnp.where(lane == i, ids, 0))
        tile_base = pl.multiple_of(lax.shift_right_logical(row_id, 7) * 128, 128)
        copies.append(
            pltpu.make_async_copy(
                tableT_hbm.at[:, pl.ds(tile_base, 128)], col_v.at[k * _CH + i], sem
            )
        )
        copies[-1].start()
    return copies


def _assemble_chunk(k, ids, marks, col_v, out_v, lane):
    offs = jnp.bitwise_and(ids, 127)
    for i in range(_CH):
        off_b = jnp.sum(jnp.where(lane == i, offs, 0)) + jnp.zeros((16,), jnp.int32)
        out_v[pl.ds((k * _CH + i) * _OUT_COLS, _EMBED)] = plsc.load_gather(
            col_v, [jnp.full((16,), k * _CH + i, jnp.int32), lane, off_b]
        )
    mcol = k * _CH * _OUT_COLS + jnp.minimum(
        lane * _OUT_COLS + _EMBED, _CH * _OUT_COLS - 1
    )
    plsc.store_scatter(out_v, [mcol], marks, mask=lane < _CH)


def _run_chunks(bases, st0_v, st1_v, null_v, col_v, out_v, tableT_hbm, out_hbm, sem):
    lane = lax.iota(jnp.int32, 16)
    null_b = jnp.sum(jnp.where(lane == 0, null_v[pl.ds(0, 16)], 0.0)) + jnp.zeros(
        (16,), jnp.float32
    )
    ids_marks = [_chunk_ids_marks(b, st0_v, st1_v, null_b, lane) for b in bases]
    copies = []
    for k, (ids, _) in enumerate(ids_marks):
        copies.extend(_fire_gathers(k, ids, col_v, tableT_hbm, sem, lane))
    for c in copies:
        c.wait()
    outs = []
    for k, (ids, marks) in enumerate(ids_marks):
        _assemble_chunk(k, ids, marks, col_v, out_v, lane)
        w = pltpu.make_async_copy(
            out_v.at[pl.ds(k * _CH * _OUT_COLS, _CH * _OUT_COLS)],
            out_hbm.at[pl.ds(bases[k] * _OUT_COLS, _CH * _OUT_COLS)],
            sem,
        )
        w.start()
        outs.append(w)
    for w in outs:
        w.wait()


def _body(st0_hbm, st1_hbm, tableT_hbm, null_hbm, out_hbm, st0_v, st1_v, null_v, col_v, out_v, sem, sem_in):
    wid = lax.axis_index("s")

    c0 = pltpu.make_async_copy(st0_hbm, st0_v.at[pl.ds(0, _SEQ)], sem_in)
    c1 = pltpu.make_async_copy(st1_hbm, st1_v.at[pl.ds(0, _SEQ)], sem_in)
    c2 = pltpu.make_async_copy(null_hbm.at[0], null_v.at[pl.ds(0, 1)], sem_in)
    c0.start()
    c1.start()
    c2.start()
    c0.wait()
    c1.wait()
    c2.wait()

    args = (st0_v, st1_v, null_v, col_v, out_v, tableT_hbm, out_hbm, sem)

    @pl.when(wid < _N_CHUNKS - _NW)
    def _():
        _run_chunks([wid * _CH, (wid + _NW) * _CH], *args)

    @pl.when(wid >= _N_CHUNKS - _NW)
    def _():
        _run_chunks([wid * _CH], *args)


@functools.partial(
    pl.kernel,
    out_type=jax.ShapeDtypeStruct((_HIST * _OUT_COLS,), jnp.float32),
    mesh=plsc.VectorSubcoreMesh(
        core_axis_name="c", subcore_axis_name="s", num_cores=1
    ),
    scratch_types=[
        pltpu.VMEM((_HIST + 8,), jnp.float32),
        pltpu.VMEM((_HIST + 8,), jnp.float32),
        pltpu.VMEM((16,), jnp.float32),
        pltpu.VMEM((2 * _CH, _EMBED, 128), jnp.float32),
        pltpu.VMEM((2 * _CH * _OUT_COLS,), jnp.float32),
        pltpu.SemaphoreType.DMA,
        pltpu.SemaphoreType.DMA,
    ],
    compiler_params=pltpu.CompilerParams(
        needs_layout_passes=False, use_tc_tiling_on_sc=True
    ),
)
def _embed_gather(st0_hbm, st1_hbm, tableT_hbm, null_hbm, out_hbm, st0_v, st1_v, null_v, col_v, out_v, sem, sem_in):
    _body(st0_hbm, st1_hbm, tableT_hbm, null_hbm, out_hbm, st0_v, st1_v, null_v, col_v, out_v, sem, sem_in)


def kernel(st, task_table, null_mark_table):
    out = _embed_gather(st[0], st[1], task_table.T, null_mark_table)
    return out.reshape(1, _HIST, _OUT_COLS)
